# baseline (speedup 1.0000x reference)
import jax
import jax.numpy as jnp
import numpy as np
from jax import lax
from jax.experimental import pallas as pl
from jax.experimental.pallas import tpu as pltpu
from jax.experimental.pallas import tpu_sc as plsc

B = 4096
L = 50
D = 32
N = B * L
M_HASH = 999999
C_A = 2654435761
C_B = 2246822519
KA = [(C_A * (1 << (8 * i))) % M_HASH for i in range(4)]
KB = [(C_B * (1 << (8 * i))) % M_HASH for i in range(4)]

NW = 32
PER_W = N // NW
CHUNK = 256
NCH = PER_W // CHUNK
KSUB = CHUNK // 128
NGR = CHUNK // 16

RS = 32
R = RS * 128


def _idiv_vec(x, d):
    q = (x.astype(jnp.float32) * (1.0 / d)).astype(jnp.int32)
    r = x - q * d
    return q + (r >= d).astype(jnp.int32) - (r < 0).astype(jnp.int32)


def _sc_body(mcc_hbm, mer_hbm, ts_hbm,
             cat_hbm, emba_hbm, embb_hbm, hr_hbm, dw_hbm, dm_hbm, out_hbm,
             mcc_v, mer_v, ts_v, ha_v, hb_v, hr_v, dw_v, dm_v, oidx_v,
             cat_rows, a_rows, b_rows, hr_rows, dw_rows, dm_rows,
             sem0, sem1, sem2, sem3):
    cid = lax.axis_index("c")
    sid = lax.axis_index("s")
    wid = sid * 2 + cid
    iota = lax.iota(jnp.int32, 16)
    zeros16 = jnp.zeros((16,), jnp.float32)

    def chunk_body(ch, carry):
        base = wid * PER_W + ch * CHUNK
        loads = []
        for j in range(KSUB):
            sl = pl.ds(base + j * 128, 128)
            jj32 = jnp.int32(j)
            loads.append(pltpu.async_copy(mcc_hbm.at[sl], mcc_v.at[jj32], sem0))
            loads.append(pltpu.async_copy(mer_hbm.at[sl], mer_v.at[jj32], sem0))
            loads.append(pltpu.async_copy(ts_hbm.at[sl], ts_v.at[jj32], sem0))
        for cp in loads:
            cp.wait()

        def comp_body(g, c2):
            jj = g >> 3
            cc = (g & 7) * 16
            m = mer_v[jj, pl.ds(cc, 16)]
            b0 = m & 0xFF
            b1 = (m >> 8) & 0xFF
            b2 = (m >> 16) & 0xFF
            b3 = (m >> 24) & 0x7F
            sa = b0 * KA[0] + b1 * KA[1] + b2 * KA[2] + b3 * KA[3]
            sb = b0 * KB[0] + b1 * KB[1] + b2 * KB[2] + b3 * KB[3]
            ha_v[jj, pl.ds(cc, 16)] = lax.rem(sa, jnp.int32(M_HASH)) + 1
            hb_v[jj, pl.ds(cc, 16)] = lax.rem(sb, jnp.int32(M_HASH)) + 1

            t = ts_v[jj, pl.ds(cc, 16)]
            d0 = _idiv_vec(t, 86400)
            r0 = t - d0 * 86400
            hr_v[jj, pl.ds(cc, 16)] = _idiv_vec(r0, 3600) + 1
            dw_v[jj, pl.ds(cc, 16)] = d0 + 3 - _idiv_vec(d0 + 3, 7) * 7 + 1
            aa = d0 + 2440588 + 32044
            bb = _idiv_vec(4 * aa + 3, 146097)
            cc2 = aa - ((146097 * bb) >> 2)
            dd = _idiv_vec(4 * cc2 + 3, 1461)
            ee = cc2 - ((1461 * dd) >> 2)
            mm = _idiv_vec(5 * ee + 2, 153)
            dm_v[jj, pl.ds(cc, 16)] = ee - _idiv_vec(153 * mm + 2, 5) + 1

            oi = (base + g * 16 + iota) * 6
            for s in range(5):
                oidx_v[s, jj, pl.ds(cc, 16)] = oi + s
            return c2
        lax.fori_loop(jnp.int32(0), jnp.int32(NGR), comp_body, jnp.int32(0))

        gathers = []
        for j in range(KSUB):
            jj32 = jnp.int32(j)
            dst = pl.ds(j * 128, 128)
            gathers.append(pltpu.async_copy(
                cat_hbm.at[mcc_v.at[jj32]], cat_rows.at[dst], sem1))
            gathers.append(pltpu.async_copy(
                emba_hbm.at[ha_v.at[jj32]], a_rows.at[dst], sem2))
            gathers.append(pltpu.async_copy(
                embb_hbm.at[hb_v.at[jj32]], b_rows.at[dst], sem3))
            gathers.append(pltpu.async_copy(
                hr_hbm.at[hr_v.at[jj32]], hr_rows.at[dst], sem1))
            gathers.append(pltpu.async_copy(
                dw_hbm.at[dw_v.at[jj32]], dw_rows.at[dst], sem2))
            gathers.append(pltpu.async_copy(
                dm_hbm.at[dm_v.at[jj32]], dm_rows.at[dst], sem3))
        for cp in gathers:
            cp.wait()

        def sum_body(g, c2):
            r = g >> 1
            cc = (g & 1) * 16
            a_rows[r, pl.ds(cc, 16)] = (
                a_rows[r, pl.ds(cc, 16)] + b_rows[r, pl.ds(cc, 16)])
            return c2
        lax.fori_loop(jnp.int32(0), jnp.int32(CHUNK * 2), sum_body, jnp.int32(0))

        def fix_body(g, c2):
            jj = g >> 3
            cc = (g & 7) * 16
            rows = g * 16 + iota
            for idx_v, targets in ((mcc_v, (cat_rows,)),
                                   (mer_v, (a_rows,)),
                                   (ts_v, (hr_rows, dw_rows, dm_rows))):
                mk = idx_v[jj, pl.ds(cc, 16)] == 0
                cnt = jnp.sum(mk.astype(jnp.int32), dtype=jnp.int32)

                @pl.when(cnt > 0)
                def _fix(targets=targets, mk=mk):
                    for rows_v in targets:
                        for col in range(D):
                            plsc.store_scatter(
                                rows_v,
                                [rows, jnp.full((16,), col, jnp.int32)],
                                zeros16, mask=mk)
            return c2
        lax.fori_loop(jnp.int32(0), jnp.int32(NGR), fix_body, jnp.int32(0))

        scats = []
        for j in range(KSUB):
            src = pl.ds(j * 128, 128)
            for s, rows_v in enumerate(
                    (cat_rows, a_rows, hr_rows, dw_rows, dm_rows)):
                scats.append(pltpu.async_copy(
                    rows_v.at[src],
                    out_hbm.at[oidx_v.at[jnp.int32(s), jnp.int32(j)]],
                    (sem1, sem2, sem3)[s % 3]))
        for cp in scats:
            cp.wait()
        return carry

    lax.fori_loop(jnp.int32(0), jnp.int32(NCH), chunk_body, jnp.int32(0))


def _tc_body(alias_ref, amt_ref, freqs_ref, wt_ref, bp_ref, out_ref):
    amt = amt_ref[...]
    fs = freqs_ref[...]
    ft_rows = []
    for k in range(16):
        ft_rows.append(jnp.sin(amt * fs[0, k]).reshape(R))
    for k in range(16):
        ft_rows.append(jnp.cos(amt * fs[0, k]).reshape(R))
    ft = jnp.stack(ft_rows, axis=0)
    num = lax.dot_general(ft, wt_ref[...], (((0,), (0,)), ((), ())),
                          preferred_element_type=jnp.float32)
    out_ref[:, 0, 0, :] = num + bp_ref[...]


def kernel(mcc, merchant, ts, amount, table_cat, emb_a, emb_b, hour_tab,
           dow_tab, dom_tab, freqs, W_proj, b_proj):
    mcc32 = mcc.reshape(N).astype(jnp.int32)
    mer32 = merchant.reshape(N).astype(jnp.int32)
    ts32 = ts.reshape(N).astype(jnp.int32)
    amt2d = amount.reshape(N // 128, 128)

    mesh = plsc.VectorSubcoreMesh(core_axis_name="c", subcore_axis_name="s")
    sc_fn = pl.kernel(
        _sc_body,
        out_type=jax.ShapeDtypeStruct((N * 6, D), jnp.float32),
        mesh=mesh,
        scratch_types=[
            pltpu.VMEM((KSUB, 128), jnp.int32),
            pltpu.VMEM((KSUB, 128), jnp.int32),
            pltpu.VMEM((KSUB, 128), jnp.int32),
            pltpu.VMEM((KSUB, 128), jnp.int32),
            pltpu.VMEM((KSUB, 128), jnp.int32),
            pltpu.VMEM((KSUB, 128), jnp.int32),
            pltpu.VMEM((KSUB, 128), jnp.int32),
            pltpu.VMEM((KSUB, 128), jnp.int32),
            pltpu.VMEM((5, KSUB, 128), jnp.int32),
            pltpu.VMEM((CHUNK, D), jnp.float32),
            pltpu.VMEM((CHUNK, D), jnp.float32),
            pltpu.VMEM((CHUNK, D), jnp.float32),
            pltpu.VMEM((CHUNK, D), jnp.float32),
            pltpu.VMEM((CHUNK, D), jnp.float32),
            pltpu.VMEM((CHUNK, D), jnp.float32),
            pltpu.SemaphoreType.DMA,
            pltpu.SemaphoreType.DMA,
            pltpu.SemaphoreType.DMA,
            pltpu.SemaphoreType.DMA,
        ],
        compiler_params=pltpu.CompilerParams(
            use_tc_tiling_on_sc=False, needs_layout_passes=False),
    )
    sc_out = sc_fn(mcc32, mer32, ts32, table_cat, emb_a, emb_b,
                   hour_tab, dow_tab, dom_tab)

    wt = W_proj.T
    freqs2 = freqs.reshape(1, 16)
    bp2 = b_proj.reshape(1, D)
    alias_in = sc_out.reshape(N, 6, 1, D)

    out4 = pl.pallas_call(
        _tc_body,
        grid=(N // R,),
        in_specs=[
            pl.BlockSpec(memory_space=pl.ANY),
            pl.BlockSpec((RS, 128), lambda i: (i, np.int32(0))),
            pl.BlockSpec((1, 16), lambda i: (np.int32(0), np.int32(0))),
            pl.BlockSpec((32, 32), lambda i: (np.int32(0), np.int32(0))),
            pl.BlockSpec((1, D), lambda i: (np.int32(0), np.int32(0))),
        ],
        out_specs=pl.BlockSpec(
            (R, 1, 1, D),
            lambda i: (i, np.int32(5), np.int32(0), np.int32(0))),
        out_shape=jax.ShapeDtypeStruct((N, 6, 1, D), jnp.float32),
        input_output_aliases={0: 0},
    )(alias_in, amt2d, freqs2, wt, bp2)

    return out4.reshape(B, L, 6, D)


# speedup vs baseline: 4.7343x; 4.7343x over previous
import jax
import jax.numpy as jnp
import numpy as np
from jax import lax
from jax.experimental import pallas as pl
from jax.experimental.pallas import tpu as pltpu
from jax.experimental.pallas import tpu_sc as plsc

B = 4096
L = 50
D = 32
N = B * L
M_HASH = 999999
C_A = 2654435761
C_B = 2246822519
KA = [(C_A * (1 << (8 * i))) % M_HASH for i in range(4)]
KB = [(C_B * (1 << (8 * i))) % M_HASH for i in range(4)]

NW = 32
PER_W = N // NW
CHUNK = 256
NCH = PER_W // CHUNK
KSUB = CHUNK // 128
NGR = CHUNK // 16


def _sc_body(mcc_hbm, mer_hbm, cat_hbm, emba_hbm, embb_hbm, out_hbm,
             mcc_v, mer_v, ha_v, hb_v,
             cat_rows, a_rows, b_rows, pad_a, pad_b,
             sem0, sem1, sem2, sem3):
    cid = lax.axis_index("c")
    sid = lax.axis_index("s")
    wid = sid * 2 + cid
    iota = lax.iota(jnp.int32, 16)
    zeros16 = jnp.zeros((16,), jnp.float32)

    def chunk_body(ch, carry):
        base = wid * PER_W + ch * CHUNK
        loads = []
        for j in range(KSUB):
            sl = pl.ds(base + j * 128, 128)
            jj32 = jnp.int32(j)
            loads.append(pltpu.async_copy(mcc_hbm.at[sl], mcc_v.at[jj32], sem0))
            loads.append(pltpu.async_copy(mer_hbm.at[sl], mer_v.at[jj32], sem0))
        for cp in loads:
            cp.wait()

        def hash_body(g, c2):
            jj = g >> 3
            cc = (g & 7) * 16
            m = mer_v[jj, pl.ds(cc, 16)]
            b0 = m & 0xFF
            b1 = (m >> 8) & 0xFF
            b2 = (m >> 16) & 0xFF
            b3 = (m >> 24) & 0x7F
            sa = b0 * KA[0] + b1 * KA[1] + b2 * KA[2] + b3 * KA[3]
            sb = b0 * KB[0] + b1 * KB[1] + b2 * KB[2] + b3 * KB[3]
            ha_v[jj, pl.ds(cc, 16)] = lax.rem(sa, jnp.int32(M_HASH)) + 1
            hb_v[jj, pl.ds(cc, 16)] = lax.rem(sb, jnp.int32(M_HASH)) + 1
            return c2
        lax.fori_loop(jnp.int32(0), jnp.int32(NGR), hash_body, jnp.int32(0))

        gathers = []
        for j in range(KSUB):
            jj32 = jnp.int32(j)
            dst = pl.ds(j * 128, 128)
            gathers.append(pltpu.async_copy(
                cat_hbm.at[mcc_v.at[jj32]], cat_rows.at[dst], sem1))
            gathers.append(pltpu.async_copy(
                emba_hbm.at[ha_v.at[jj32]], a_rows.at[dst], sem2))
            gathers.append(pltpu.async_copy(
                embb_hbm.at[hb_v.at[jj32]], b_rows.at[dst], sem3))
        for cp in gathers:
            cp.wait()

        def pack_body(g, c2):
            r = g >> 1
            cc = (g & 1) * 16
            pad_a[r, pl.ds(cc, 16)] = cat_rows[r, pl.ds(cc, 16)]
            pad_b[r, pl.ds(cc, 16)] = (
                a_rows[r, pl.ds(cc, 16)] + b_rows[r, pl.ds(cc, 16)])
            return c2
        lax.fori_loop(jnp.int32(0), jnp.int32(CHUNK * 2), pack_body, jnp.int32(0))

        def fix_body(g, c2):
            jj = g >> 3
            cc = (g & 7) * 16
            rows = g * 16 + iota
            for idx_v, rows_v in ((mcc_v, pad_a), (mer_v, pad_b)):
                mk = idx_v[jj, pl.ds(cc, 16)] == 0
                cnt = jnp.sum(mk.astype(jnp.int32), dtype=jnp.int32)

                @pl.when(cnt > 0)
                def _fix(rows_v=rows_v, mk=mk):
                    for col in range(D):
                        plsc.store_scatter(
                            rows_v,
                            [rows, jnp.full((16,), col, jnp.int32)],
                            zeros16, mask=mk)
            return c2
        lax.fori_loop(jnp.int32(0), jnp.int32(NGR), fix_body, jnp.int32(0))

        s0 = pltpu.async_copy(pad_a, out_hbm.at[jnp.int32(0), pl.ds(base, CHUNK)], sem1)
        s1 = pltpu.async_copy(pad_b, out_hbm.at[jnp.int32(1), pl.ds(base, CHUNK)], sem2)
        s0.wait()
        s1.wait()
        return carry

    lax.fori_loop(jnp.int32(0), jnp.int32(NCH), chunk_body, jnp.int32(0))


def _idiv(x, d):
    q = (x.astype(jnp.float32) * (1.0 / d)).astype(jnp.int32)
    r = x - q * d
    return q + (r >= d).astype(jnp.int32) - (r < 0).astype(jnp.int32)


def _onehot_t(idx, nv):
    rows = [(idx == (v + 1)).astype(jnp.float32).reshape(B) for v in range(nv)]
    return jnp.stack(rows, axis=0)


def _tc_body(sc_ref, ts_ref, amt_ref, htabT_ref, dwtabT_ref, dmtabT_ref,
             freqs_ref, w_ref, bp_ref, out_ref):
    xa = sc_ref[0, 0]
    xb = sc_ref[1, 0]
    out_ref[0, 0] = lax.transpose(xa, (1, 0))[0:D, :]
    out_ref[0, 1] = lax.transpose(xb, (1, 0))[0:D, :]

    t32 = ts_ref[0]
    d0 = _idiv(t32, 86400)
    r0 = t32 - d0 * 86400
    hour = jnp.where(t32 == 0, 0, _idiv(r0, 3600) + 1)
    dow = jnp.where(t32 == 0, 0, d0 + 3 - _idiv(d0 + 3, 7) * 7 + 1)
    aa = d0 + 2440588 + 32044
    bb = _idiv(4 * aa + 3, 146097)
    cc = aa - ((146097 * bb) >> 2)
    dd = _idiv(4 * cc + 3, 1461)
    ee = cc - ((1461 * dd) >> 2)
    mm = _idiv(5 * ee + 2, 153)
    dom = jnp.where(t32 == 0, 0, ee - _idiv(153 * mm + 2, 5) + 1)
    dn = (((1,), (0,)), ((), ()))
    out_ref[0, 2] = lax.dot_general(htabT_ref[...], _onehot_t(hour, 24), dn,
                                    preferred_element_type=jnp.float32)
    out_ref[0, 3] = lax.dot_general(dwtabT_ref[...], _onehot_t(dow, 7), dn,
                                    preferred_element_type=jnp.float32)
    out_ref[0, 4] = lax.dot_general(dmtabT_ref[...], _onehot_t(dom, 31), dn,
                                    preferred_element_type=jnp.float32)

    amt = amt_ref[0]
    fs = freqs_ref[...]
    ft_rows = []
    for k in range(16):
        ft_rows.append(jnp.sin(amt * fs[0, k]).reshape(B))
    for k in range(16):
        ft_rows.append(jnp.cos(amt * fs[0, k]).reshape(B))
    ft = jnp.stack(ft_rows, axis=0)
    num_t = lax.dot_general(w_ref[...], ft, dn,
                            preferred_element_type=jnp.float32)
    out_ref[0, 5] = num_t + bp_ref[...]


def kernel(mcc, merchant, ts, amount, table_cat, emb_a, emb_b, hour_tab,
           dow_tab, dom_tab, freqs, W_proj, b_proj):
    mccT = mcc.T.reshape(N).astype(jnp.int32)
    merT = merchant.T.reshape(N).astype(jnp.int32)
    ts3 = ts.T.reshape(L, D, 128).astype(jnp.int32)
    amt3 = amount.T.reshape(L, D, 128)

    mesh = plsc.VectorSubcoreMesh(core_axis_name="c", subcore_axis_name="s")
    sc_fn = pl.kernel(
        _sc_body,
        out_type=jax.ShapeDtypeStruct((2, N, 128), jnp.float32),
        mesh=mesh,
        scratch_types=[
            pltpu.VMEM((KSUB, 128), jnp.int32),
            pltpu.VMEM((KSUB, 128), jnp.int32),
            pltpu.VMEM((KSUB, 128), jnp.int32),
            pltpu.VMEM((KSUB, 128), jnp.int32),
            pltpu.VMEM((CHUNK, D), jnp.float32),
            pltpu.VMEM((CHUNK, D), jnp.float32),
            pltpu.VMEM((CHUNK, D), jnp.float32),
            pltpu.VMEM((CHUNK, 128), jnp.float32),
            pltpu.VMEM((CHUNK, 128), jnp.float32),
            pltpu.SemaphoreType.DMA,
            pltpu.SemaphoreType.DMA,
            pltpu.SemaphoreType.DMA,
            pltpu.SemaphoreType.DMA,
        ],
        compiler_params=pltpu.CompilerParams(
            use_tc_tiling_on_sc=False, needs_layout_passes=False),
    )
    sc_out = sc_fn(mccT, merT, table_cat, emb_a, emb_b)
    sc4 = sc_out.reshape(2, L, B, 128)

    htabT = hour_tab[1:25].T
    dwtabT = dow_tab[1:8].T
    dmtabT = dom_tab[1:32].T
    freqs2 = freqs.reshape(1, 16)
    bp2 = b_proj.reshape(D, 1)

    z = np.int32(0)
    out_t = pl.pallas_call(
        _tc_body,
        grid=(L,),
        in_specs=[
            pl.BlockSpec((2, 1, B, 128), lambda i: (z, i, z, z)),
            pl.BlockSpec((1, D, 128), lambda i: (i, z, z)),
            pl.BlockSpec((1, D, 128), lambda i: (i, z, z)),
            pl.BlockSpec((D, 24), lambda i: (z, z)),
            pl.BlockSpec((D, 7), lambda i: (z, z)),
            pl.BlockSpec((D, 31), lambda i: (z, z)),
            pl.BlockSpec((1, 16), lambda i: (z, z)),
            pl.BlockSpec((D, D), lambda i: (z, z)),
            pl.BlockSpec((D, 1), lambda i: (z, z)),
        ],
        out_specs=pl.BlockSpec((1, 6, D, B), lambda i: (i, z, z, z)),
        out_shape=jax.ShapeDtypeStruct((L, 6, D, B), jnp.float32),
    )(sc4, ts3, amt3, htabT, dwtabT, dmtabT, freqs2, W_proj, bp2)

    return jnp.transpose(out_t, (3, 0, 1, 2))


# speedup vs baseline: 4.8611x; 1.0268x over previous
import jax
import jax.numpy as jnp
import numpy as np
from jax import lax
from jax.experimental import pallas as pl
from jax.experimental.pallas import tpu as pltpu
from jax.experimental.pallas import tpu_sc as plsc

B = 4096
L = 50
D = 32
N = B * L
M_HASH = 999999
C_A = 2654435761
C_B = 2246822519
KA = [(C_A * (1 << (8 * i))) % M_HASH for i in range(4)]
KB = [(C_B * (1 << (8 * i))) % M_HASH for i in range(4)]

NW = 32
PER_W = N // NW
CHUNK = 256
NCH = PER_W // CHUNK
KSUB = CHUNK // 128
NGR = CHUNK // 16


def _sc_body(mcc_hbm, mer_hbm, cat_hbm, emba_hbm, embb_hbm, out_hbm,
             mcc0, mer0, ha0, hb0, cat0, arow0, brow0,
             mcc1, mer1, ha1, hb1, cat1, arow1, brow1,
             pad_a, pad_b,
             sem_l0, sem_l1, sem_g0, sem_g1, sem_s):
    cid = lax.axis_index("c")
    sid = lax.axis_index("s")
    wid = sid * 2 + cid
    iota = lax.iota(jnp.int32, 16)
    zeros16 = jnp.zeros((16,), jnp.float32)
    bufs = ((mcc0, mer0, ha0, hb0, cat0, arow0, brow0, sem_l0, sem_g0),
            (mcc1, mer1, ha1, hb1, cat1, arow1, brow1, sem_l1, sem_g1))

    def produce(ck, par):
        mcc_v, mer_v, ha_v, hb_v, cat_r, a_r, b_r, sem_l, sem_g = bufs[par]
        base = wid * PER_W + ck * CHUNK
        for j in range(KSUB):
            sl = pl.ds(base + j * 128, 128)
            jj32 = jnp.int32(j)
            pltpu.async_copy(mcc_hbm.at[sl], mcc_v.at[jj32], sem_l)
            pltpu.async_copy(mer_hbm.at[sl], mer_v.at[jj32], sem_l)
        for j in range(KSUB):
            sl = pl.ds(base + j * 128, 128)
            jj32 = jnp.int32(j)
            pltpu.make_async_copy(mcc_hbm.at[sl], mcc_v.at[jj32], sem_l).wait()
            pltpu.make_async_copy(mer_hbm.at[sl], mer_v.at[jj32], sem_l).wait()

        def hash_body(g, c2):
            jj = g >> 3
            cc = (g & 7) * 16
            m = mer_v[jj, pl.ds(cc, 16)]
            b0 = m & 0xFF
            b1 = (m >> 8) & 0xFF
            b2 = (m >> 16) & 0xFF
            b3 = (m >> 24) & 0x7F
            sa = b0 * KA[0] + b1 * KA[1] + b2 * KA[2] + b3 * KA[3]
            sb = b0 * KB[0] + b1 * KB[1] + b2 * KB[2] + b3 * KB[3]
            ha_v[jj, pl.ds(cc, 16)] = lax.rem(sa, jnp.int32(M_HASH)) + 1
            hb_v[jj, pl.ds(cc, 16)] = lax.rem(sb, jnp.int32(M_HASH)) + 1
            return c2
        lax.fori_loop(jnp.int32(0), jnp.int32(NGR), hash_body, jnp.int32(0))

        for j in range(KSUB):
            jj32 = jnp.int32(j)
            dst = pl.ds(j * 128, 128)
            pltpu.async_copy(cat_hbm.at[mcc_v.at[jj32]], cat_r.at[dst], sem_g)
            pltpu.async_copy(emba_hbm.at[ha_v.at[jj32]], a_r.at[dst], sem_g)
            pltpu.async_copy(embb_hbm.at[hb_v.at[jj32]], b_r.at[dst], sem_g)

    def consume(ck, par):
        mcc_v, mer_v, ha_v, hb_v, cat_r, a_r, b_r, sem_l, sem_g = bufs[par]
        base = wid * PER_W + ck * CHUNK
        for j in range(KSUB):
            dst = pl.ds(j * 128, 128)
            pltpu.make_async_copy(
                cat_hbm.at[pl.ds(0, 128)], cat_r.at[dst], sem_g).wait()
            pltpu.make_async_copy(
                emba_hbm.at[pl.ds(0, 128)], a_r.at[dst], sem_g).wait()
            pltpu.make_async_copy(
                embb_hbm.at[pl.ds(0, 128)], b_r.at[dst], sem_g).wait()

        def pack_body(g, c2):
            r = g >> 1
            cc = (g & 1) * 16
            pad_a[r, pl.ds(cc, 16)] = cat_r[r, pl.ds(cc, 16)]
            pad_b[r, pl.ds(cc, 16)] = (
                a_r[r, pl.ds(cc, 16)] + b_r[r, pl.ds(cc, 16)])
            return c2
        lax.fori_loop(jnp.int32(0), jnp.int32(CHUNK * 2), pack_body,
                      jnp.int32(0))

        def fix_body(g, c2):
            jj = g >> 3
            cc = (g & 7) * 16
            rows = g * 16 + iota
            for idx_v, rows_v in ((mcc_v, pad_a), (mer_v, pad_b)):
                mk = idx_v[jj, pl.ds(cc, 16)] == 0
                cnt = jnp.sum(mk.astype(jnp.int32), dtype=jnp.int32)

                @pl.when(cnt > 0)
                def _fix(rows_v=rows_v, mk=mk):
                    for col in range(D):
                        plsc.store_scatter(
                            rows_v,
                            [rows, jnp.full((16,), col, jnp.int32)],
                            zeros16, mask=mk)
            return c2
        lax.fori_loop(jnp.int32(0), jnp.int32(NGR), fix_body, jnp.int32(0))

        s0 = pltpu.async_copy(
            pad_a, out_hbm.at[jnp.int32(0), pl.ds(base, CHUNK)], sem_s)
        s1 = pltpu.async_copy(
            pad_b, out_hbm.at[jnp.int32(1), pl.ds(base, CHUNK)], sem_s)
        s0.wait()
        s1.wait()

    produce(jnp.int32(0), 0)

    def super_body(h, carry):
        k0 = h * 2
        produce(k0 + 1, 1)
        consume(k0, 0)
        produce(k0 + 2, 0)
        consume(k0 + 1, 1)
        return carry
    lax.fori_loop(jnp.int32(0), jnp.int32(NCH // 2), super_body, jnp.int32(0))
    consume(jnp.int32(NCH - 1), 0)


def _idiv(x, d):
    q = (x.astype(jnp.float32) * (1.0 / d)).astype(jnp.int32)
    r = x - q * d
    return q + (r >= d).astype(jnp.int32) - (r < 0).astype(jnp.int32)


def _onehot_t(idx, nv):
    rows = [(idx == (v + 1)).astype(jnp.float32).reshape(B) for v in range(nv)]
    return jnp.stack(rows, axis=0)


def _tc_body(sc_ref, ts_ref, amt_ref, htabT_ref, dwtabT_ref, dmtabT_ref,
             freqs_ref, w_ref, bp_ref, out_ref):
    xa = sc_ref[0, 0]
    xb = sc_ref[1, 0]
    out_ref[0, 0] = lax.transpose(xa, (1, 0))[0:D, :]
    out_ref[0, 1] = lax.transpose(xb, (1, 0))[0:D, :]

    t32 = ts_ref[0]
    d0 = _idiv(t32, 86400)
    r0 = t32 - d0 * 86400
    hour = jnp.where(t32 == 0, 0, _idiv(r0, 3600) + 1)
    dow = jnp.where(t32 == 0, 0, d0 + 3 - _idiv(d0 + 3, 7) * 7 + 1)
    aa = d0 + 2440588 + 32044
    bb = _idiv(4 * aa + 3, 146097)
    cc = aa - ((146097 * bb) >> 2)
    dd = _idiv(4 * cc + 3, 1461)
    ee = cc - ((1461 * dd) >> 2)
    mm = _idiv(5 * ee + 2, 153)
    dom = jnp.where(t32 == 0, 0, ee - _idiv(153 * mm + 2, 5) + 1)
    dn = (((1,), (0,)), ((), ()))
    out_ref[0, 2] = lax.dot_general(htabT_ref[...], _onehot_t(hour, 24), dn,
                                    preferred_element_type=jnp.float32)
    out_ref[0, 3] = lax.dot_general(dwtabT_ref[...], _onehot_t(dow, 7), dn,
                                    preferred_element_type=jnp.float32)
    out_ref[0, 4] = lax.dot_general(dmtabT_ref[...], _onehot_t(dom, 31), dn,
                                    preferred_element_type=jnp.float32)

    amt = amt_ref[0]
    fs = freqs_ref[...]
    ft_rows = []
    for k in range(16):
        ft_rows.append(jnp.sin(amt * fs[0, k]).reshape(B))
    for k in range(16):
        ft_rows.append(jnp.cos(amt * fs[0, k]).reshape(B))
    ft = jnp.stack(ft_rows, axis=0)
    num_t = lax.dot_general(w_ref[...], ft, dn,
                            preferred_element_type=jnp.float32)
    out_ref[0, 5] = num_t + bp_ref[...]


def kernel(mcc, merchant, ts, amount, table_cat, emb_a, emb_b, hour_tab,
           dow_tab, dom_tab, freqs, W_proj, b_proj):
    mccT = mcc.T.reshape(N).astype(jnp.int32)
    merT = merchant.T.reshape(N).astype(jnp.int32)
    ts3 = ts.T.reshape(L, D, 128).astype(jnp.int32)
    amt3 = amount.T.reshape(L, D, 128)

    mesh = plsc.VectorSubcoreMesh(core_axis_name="c", subcore_axis_name="s")
    sc_fn = pl.kernel(
        _sc_body,
        out_type=jax.ShapeDtypeStruct((2, N, 128), jnp.float32),
        mesh=mesh,
        scratch_types=(
            [pltpu.VMEM((KSUB, 128), jnp.int32)] * 4
            + [pltpu.VMEM((CHUNK, D), jnp.float32)] * 3
            + [pltpu.VMEM((KSUB, 128), jnp.int32)] * 4
            + [pltpu.VMEM((CHUNK, D), jnp.float32)] * 3
            + [pltpu.VMEM((CHUNK, 128), jnp.float32)] * 2
            + [pltpu.SemaphoreType.DMA] * 5
        ),
        compiler_params=pltpu.CompilerParams(
            use_tc_tiling_on_sc=False, needs_layout_passes=False),
    )
    sc_out = sc_fn(mccT, merT, table_cat, emb_a, emb_b)
    sc4 = sc_out.reshape(2, L, B, 128)

    htabT = hour_tab[1:25].T
    dwtabT = dow_tab[1:8].T
    dmtabT = dom_tab[1:32].T
    freqs2 = freqs.reshape(1, 16)
    bp2 = b_proj.reshape(D, 1)

    z = np.int32(0)
    out_t = pl.pallas_call(
        _tc_body,
        grid=(L,),
        in_specs=[
            pl.BlockSpec((2, 1, B, 128), lambda i: (z, i, z, z)),
            pl.BlockSpec((1, D, 128), lambda i: (i, z, z)),
            pl.BlockSpec((1, D, 128), lambda i: (i, z, z)),
            pl.BlockSpec((D, 24), lambda i: (z, z)),
            pl.BlockSpec((D, 7), lambda i: (z, z)),
            pl.BlockSpec((D, 31), lambda i: (z, z)),
            pl.BlockSpec((1, 16), lambda i: (z, z)),
            pl.BlockSpec((D, D), lambda i: (z, z)),
            pl.BlockSpec((D, 1), lambda i: (z, z)),
        ],
        out_specs=pl.BlockSpec((1, 6, D, B), lambda i: (i, z, z, z)),
        out_shape=jax.ShapeDtypeStruct((L, 6, D, B), jnp.float32),
    )(sc4, ts3, amt3, htabT, dwtabT, dmtabT, freqs2, W_proj, bp2)

    return jnp.transpose(out_t, (3, 0, 1, 2))


# speedup vs baseline: 5.1174x; 1.0527x over previous
import jax
import jax.numpy as jnp
import numpy as np
from jax import lax
from jax.experimental import pallas as pl
from jax.experimental.pallas import tpu as pltpu
from jax.experimental.pallas import tpu_sc as plsc

B = 4096
L = 50
D = 32
N = B * L
M_HASH = 999999
C_A = 2654435761
C_B = 2246822519
KA = [(C_A * (1 << (8 * i))) % M_HASH for i in range(4)]
KB = [(C_B * (1 << (8 * i))) % M_HASH for i in range(4)]

NW = 32
PER_W = N // NW
CHUNK = 256
NCH = PER_W // CHUNK
KSUB = CHUNK // 128
NGR = CHUNK // 16


def _sc_body(mcc_hbm, mer_hbm, cat_hbm, emba_hbm, embb_hbm, out_hbm,
             mcc0, mer0, ha0, hb0, cat0, arow0, brow0,
             mcc1, mer1, ha1, hb1, cat1, arow1, brow1,
             pad_a, pad_b,
             sem_l0, sem_l1, sem_g0, sem_g1, sem_s):
    cid = lax.axis_index("c")
    sid = lax.axis_index("s")
    wid = sid * 2 + cid
    iota = lax.iota(jnp.int32, 16)
    zeros16 = jnp.zeros((16,), jnp.float32)
    bufs = ((mcc0, mer0, ha0, hb0, cat0, arow0, brow0, sem_l0, sem_g0),
            (mcc1, mer1, ha1, hb1, cat1, arow1, brow1, sem_l1, sem_g1))

    def produce(ck, par):
        mcc_v, mer_v, ha_v, hb_v, cat_r, a_r, b_r, sem_l, sem_g = bufs[par]
        base = wid * PER_W + ck * CHUNK
        for j in range(KSUB):
            sl = pl.ds(base + j * 128, 128)
            jj32 = jnp.int32(j)
            pltpu.async_copy(mcc_hbm.at[sl], mcc_v.at[jj32], sem_l)
            pltpu.async_copy(mer_hbm.at[sl], mer_v.at[jj32], sem_l)
        for j in range(KSUB):
            sl = pl.ds(base + j * 128, 128)
            jj32 = jnp.int32(j)
            pltpu.make_async_copy(mcc_hbm.at[sl], mcc_v.at[jj32], sem_l).wait()
            pltpu.make_async_copy(mer_hbm.at[sl], mer_v.at[jj32], sem_l).wait()

        def hash_body(g, c2):
            jj = g >> 3
            cc = (g & 7) * 16
            m = mer_v[jj, pl.ds(cc, 16)]
            b0 = m & 0xFF
            b1 = (m >> 8) & 0xFF
            b2 = (m >> 16) & 0xFF
            b3 = (m >> 24) & 0x7F
            sa = b0 * KA[0] + b1 * KA[1] + b2 * KA[2] + b3 * KA[3]
            sb = b0 * KB[0] + b1 * KB[1] + b2 * KB[2] + b3 * KB[3]
            ha_v[jj, pl.ds(cc, 16)] = lax.rem(sa, jnp.int32(M_HASH)) + 1
            hb_v[jj, pl.ds(cc, 16)] = lax.rem(sb, jnp.int32(M_HASH)) + 1
            return c2
        lax.fori_loop(jnp.int32(0), jnp.int32(NGR), hash_body, jnp.int32(0))

        for j in range(KSUB):
            jj32 = jnp.int32(j)
            dst = pl.ds(j * 128, 128)
            pltpu.async_copy(cat_hbm.at[mcc_v.at[jj32]], cat_r.at[dst], sem_g)
            pltpu.async_copy(emba_hbm.at[ha_v.at[jj32]], a_r.at[dst], sem_g)
            pltpu.async_copy(embb_hbm.at[hb_v.at[jj32]], b_r.at[dst], sem_g)

    def consume(ck, par):
        mcc_v, mer_v, ha_v, hb_v, cat_r, a_r, b_r, sem_l, sem_g = bufs[par]
        base = wid * PER_W + ck * CHUNK
        for j in range(KSUB):
            dst = pl.ds(j * 128, 128)
            pltpu.make_async_copy(
                cat_hbm.at[pl.ds(0, 128)], cat_r.at[dst], sem_g).wait()
            pltpu.make_async_copy(
                emba_hbm.at[pl.ds(0, 128)], a_r.at[dst], sem_g).wait()
            pltpu.make_async_copy(
                embb_hbm.at[pl.ds(0, 128)], b_r.at[dst], sem_g).wait()

        def pack_body(g, c2):
            r = g >> 1
            cc = (g & 1) * 16
            pad_a[r, pl.ds(cc, 16)] = cat_r[r, pl.ds(cc, 16)]
            pad_b[r, pl.ds(cc, 16)] = (
                a_r[r, pl.ds(cc, 16)] + b_r[r, pl.ds(cc, 16)])
            return c2
        lax.fori_loop(jnp.int32(0), jnp.int32(CHUNK * 2), pack_body,
                      jnp.int32(0))

        def fix_body(g, c2):
            jj = g >> 3
            cc = (g & 7) * 16
            rows = g * 16 + iota
            for idx_v, rows_v in ((mcc_v, pad_a), (mer_v, pad_b)):
                mk = idx_v[jj, pl.ds(cc, 16)] == 0
                cnt = jnp.sum(mk.astype(jnp.int32), dtype=jnp.int32)

                @pl.when(cnt > 0)
                def _fix(rows_v=rows_v, mk=mk):
                    for col in range(D):
                        plsc.store_scatter(
                            rows_v,
                            [rows, jnp.full((16,), col, jnp.int32)],
                            zeros16, mask=mk)
            return c2
        lax.fori_loop(jnp.int32(0), jnp.int32(NGR), fix_body, jnp.int32(0))

        s0 = pltpu.async_copy(
            pad_a, out_hbm.at[jnp.int32(0), pl.ds(base, CHUNK)], sem_s)
        s1 = pltpu.async_copy(
            pad_b, out_hbm.at[jnp.int32(1), pl.ds(base, CHUNK)], sem_s)
        s0.wait()
        s1.wait()

    produce(jnp.int32(0), 0)

    def super_body(h, carry):
        k0 = h * 2
        produce(k0 + 1, 1)
        consume(k0, 0)
        produce(k0 + 2, 0)
        consume(k0 + 1, 1)
        return carry
    lax.fori_loop(jnp.int32(0), jnp.int32(NCH // 2), super_body, jnp.int32(0))
    consume(jnp.int32(NCH - 1), 0)


def _idiv(x, d):
    q = (x.astype(jnp.float32) * (1.0 / d)).astype(jnp.int32)
    r = x - q * d
    return q + (r >= d).astype(jnp.int32) - (r < 0).astype(jnp.int32)


def _onehot_t(idx, nv):
    rows = [(idx == (v + 1)).astype(jnp.float32).reshape(B) for v in range(nv)]
    return jnp.stack(rows, axis=0)


def _tc_indep_body(ts_ref, amt_ref, htabT_ref, dwtabT_ref, dmtabT_ref,
                   freqs_ref, w_ref, bp_ref, out_ref):
    j = pl.program_id(1)

    @pl.when(j == 0)
    def _datetime():
        t32 = ts_ref[0]
        d0 = _idiv(t32, 86400)
        r0 = t32 - d0 * 86400
        hour = jnp.where(t32 == 0, 0, _idiv(r0, 3600) + 1)
        dow = jnp.where(t32 == 0, 0, d0 + 3 - _idiv(d0 + 3, 7) * 7 + 1)
        dn = (((1,), (0,)), ((), ()))
        out_ref[0, 0] = lax.dot_general(htabT_ref[...], _onehot_t(hour, 24),
                                        dn, preferred_element_type=jnp.float32)
        out_ref[0, 1] = lax.dot_general(dwtabT_ref[...], _onehot_t(dow, 7),
                                        dn, preferred_element_type=jnp.float32)

    @pl.when(j == 1)
    def _dom_num():
        t32 = ts_ref[0]
        d0 = _idiv(t32, 86400)
        aa = d0 + 2440588 + 32044
        bb = _idiv(4 * aa + 3, 146097)
        cc = aa - ((146097 * bb) >> 2)
        dd = _idiv(4 * cc + 3, 1461)
        ee = cc - ((1461 * dd) >> 2)
        mm = _idiv(5 * ee + 2, 153)
        dom = jnp.where(t32 == 0, 0, ee - _idiv(153 * mm + 2, 5) + 1)
        dn = (((1,), (0,)), ((), ()))
        out_ref[0, 0] = lax.dot_general(dmtabT_ref[...], _onehot_t(dom, 31),
                                        dn, preferred_element_type=jnp.float32)
        amt = amt_ref[0]
        fs = freqs_ref[...]
        ft_rows = []
        for k in range(16):
            ft_rows.append(jnp.sin(amt * fs[0, k]).reshape(B))
        for k in range(16):
            ft_rows.append(jnp.cos(amt * fs[0, k]).reshape(B))
        ft = jnp.stack(ft_rows, axis=0)
        num_t = lax.dot_general(w_ref[...], ft, dn,
                                preferred_element_type=jnp.float32)
        out_ref[0, 1] = num_t + bp_ref[...]


def _tc_gather_body(alias_ref, sc_ref, out_ref):
    xa = sc_ref[0, 0]
    xb = sc_ref[1, 0]
    out_ref[0, 0] = lax.transpose(xa, (1, 0))[0:D, :]
    out_ref[0, 1] = lax.transpose(xb, (1, 0))[0:D, :]


def kernel(mcc, merchant, ts, amount, table_cat, emb_a, emb_b, hour_tab,
           dow_tab, dom_tab, freqs, W_proj, b_proj):
    mccT = mcc.T.reshape(N).astype(jnp.int32)
    merT = merchant.T.reshape(N).astype(jnp.int32)
    ts3 = ts.T.reshape(L, D, 128).astype(jnp.int32)
    amt3 = amount.T.reshape(L, D, 128)

    mesh = plsc.VectorSubcoreMesh(core_axis_name="c", subcore_axis_name="s")
    sc_fn = pl.kernel(
        _sc_body,
        out_type=jax.ShapeDtypeStruct((2, N, 128), jnp.float32),
        mesh=mesh,
        scratch_types=(
            [pltpu.VMEM((KSUB, 128), jnp.int32)] * 4
            + [pltpu.VMEM((CHUNK, D), jnp.float32)] * 3
            + [pltpu.VMEM((KSUB, 128), jnp.int32)] * 4
            + [pltpu.VMEM((CHUNK, D), jnp.float32)] * 3
            + [pltpu.VMEM((CHUNK, 128), jnp.float32)] * 2
            + [pltpu.SemaphoreType.DMA] * 5
        ),
        compiler_params=pltpu.CompilerParams(
            use_tc_tiling_on_sc=False, needs_layout_passes=False),
    )
    sc_out = sc_fn(mccT, merT, table_cat, emb_a, emb_b)
    sc4 = sc_out.reshape(2, L, B, 128)

    htabT = hour_tab[1:25].T
    dwtabT = dow_tab[1:8].T
    dmtabT = dom_tab[1:32].T
    freqs2 = freqs.reshape(1, 16)
    bp2 = b_proj.reshape(D, 1)

    z = np.int32(0)
    part = pl.pallas_call(
        _tc_indep_body,
        grid=(L, 2),
        in_specs=[
            pl.BlockSpec((1, D, 128), lambda i, j: (i, z, z)),
            pl.BlockSpec((1, D, 128), lambda i, j: (i, z, z)),
            pl.BlockSpec((D, 24), lambda i, j: (z, z)),
            pl.BlockSpec((D, 7), lambda i, j: (z, z)),
            pl.BlockSpec((D, 31), lambda i, j: (z, z)),
            pl.BlockSpec((1, 16), lambda i, j: (z, z)),
            pl.BlockSpec((D, D), lambda i, j: (z, z)),
            pl.BlockSpec((D, 1), lambda i, j: (z, z)),
        ],
        out_specs=pl.BlockSpec((1, 2, D, B),
                               lambda i, j: (i, j + np.int32(1), z, z)),
        out_shape=jax.ShapeDtypeStruct((L, 6, D, B), jnp.float32),
    )(ts3, amt3, htabT, dwtabT, dmtabT, freqs2, W_proj, bp2)

    out_t = pl.pallas_call(
        _tc_gather_body,
        grid=(L,),
        in_specs=[
            pl.BlockSpec(memory_space=pl.ANY),
            pl.BlockSpec((2, 1, B, 128), lambda i: (z, i, z, z)),
        ],
        out_specs=pl.BlockSpec((1, 2, D, B), lambda i: (i, z, z, z)),
        out_shape=jax.ShapeDtypeStruct((L, 6, D, B), jnp.float32),
        input_output_aliases={0: 0},
    )(part, sc4)

    return jnp.transpose(out_t, (3, 0, 1, 2))
